# baseline (speedup 1.0000x reference)
import functools

import jax
import jax.numpy as jnp
import numpy as np
from jax.experimental import pallas as pl
from jax.experimental.pallas import tpu as pltpu

ROWS = 128
COLS = 100000
BLOCK_COLS = 2048

_TINY = np.float32(np.finfo(np.float32).tiny)
_KS0 = np.uint32(0)
_KS1 = np.uint32(42)
_KS2 = np.uint32(_KS0 ^ _KS1 ^ np.uint32(0x1BD11BDA))
_KS = (_KS0, _KS1, _KS2)
_ROTATIONS = ((13, 15, 26, 6), (17, 29, 16, 24))


def _rotl(x, d):
    return (x << np.uint32(d)) | (x >> np.uint32(32 - d))


def _threefry2x32(x0, x1):
    x0 = x0 + _KS0
    x1 = x1 + _KS1
    for i in range(5):
        for r in _ROTATIONS[i % 2]:
            x0 = x0 + x1
            x1 = _rotl(x1, r)
            x1 = x1 ^ x0
        x0 = x0 + _KS[(i + 1) % 3]
        x1 = x1 + _KS[(i + 2) % 3] + np.uint32(i + 1)
    return x0, x1


def _sample_kernel(logits_ref, out_ref, best_val, best_idx):
    j = pl.program_id(0)
    nblocks = pl.num_programs(0)

    col = j * BLOCK_COLS + jax.lax.broadcasted_iota(
        jnp.int32, (ROWS, BLOCK_COLS), 1)
    row = jax.lax.broadcasted_iota(jnp.int32, (ROWS, BLOCK_COLS), 0)
    n = (row * COLS + col).astype(jnp.uint32)

    x0, x1 = _threefry2x32(jnp.zeros_like(n), n)
    bits = x0 ^ x1

    fbits = (bits >> np.uint32(9)) | np.uint32(0x3F800000)
    floats = jax.lax.bitcast_convert_type(fbits, jnp.float32) - np.float32(1.0)
    u = jnp.maximum(_TINY, floats * (np.float32(1.0) - _TINY) + _TINY)
    g = -jnp.log(-jnp.log(u))

    x = logits_ref[...] + g
    x = jnp.where(col < COLS, x, -jnp.inf)

    bm = jnp.max(x, axis=1, keepdims=True)
    bi = jnp.min(jnp.where(x == bm, col, jnp.int32(2**30)), axis=1,
                 keepdims=True)

    @pl.when(j == 0)
    def _():
        best_val[...] = bm
        best_idx[...] = bi

    @pl.when(j > 0)
    def _():
        upd = bm > best_val[...]
        best_val[...] = jnp.where(upd, bm, best_val[...])
        best_idx[...] = jnp.where(upd, bi, best_idx[...])

    @pl.when(j == nblocks - 1)
    def _():
        out_ref[...] = best_idx[...]


@jax.jit
def kernel(logits):
    nblocks = pl.cdiv(COLS, BLOCK_COLS)
    out = pl.pallas_call(
        _sample_kernel,
        grid=(nblocks,),
        in_specs=[pl.BlockSpec((ROWS, BLOCK_COLS), lambda j: (0, j))],
        out_specs=pl.BlockSpec((ROWS, 1), lambda j: (0, 0)),
        out_shape=jax.ShapeDtypeStruct((ROWS, 1), jnp.int32),
        scratch_shapes=[
            pltpu.VMEM((ROWS, 1), jnp.float32),
            pltpu.VMEM((ROWS, 1), jnp.int32),
        ],
    )(logits)
    return out.reshape(ROWS)
